# baseline (speedup 1.0000x reference)
import functools

import jax
import jax.numpy as jnp
from jax import lax
from jax.experimental import pallas as pl
from jax.experimental.pallas import tpu as pltpu
from jax.experimental.pallas import tpu_sc as plsc

NUM_CORES = 2
NUM_SUBCORES = 16
NUM_WORKERS = NUM_CORES * NUM_SUBCORES
NBUF = 2
L = 16
DSZ = 64


@functools.lru_cache(maxsize=None)
def _make_lookup(batch: int, vhalf: int, chunk: int):
    assert batch % NUM_WORKERS == 0
    b_per_w = batch // NUM_WORKERS
    assert b_per_w % (chunk * NBUF) == 0
    n_outer = b_per_w // (chunk * NBUF)

    mesh = plsc.VectorSubcoreMesh(core_axis_name="c", subcore_axis_name="s")

    @functools.partial(
        pl.kernel,
        mesh=mesh,
        out_type=jax.ShapeDtypeStruct((batch, DSZ), jnp.float32),
        scratch_types=(
            [pltpu.VMEM((b_per_w,), jnp.int32)]
            + [pltpu.VMEM((chunk,), jnp.int32) for _ in range(NBUF)]
            + [pltpu.VMEM((chunk,), jnp.int32) for _ in range(NBUF)]
            + [pltpu.VMEM((chunk, 2 * DSZ), jnp.float32) for _ in range(NBUF)]
            + [pltpu.VMEM((chunk, DSZ), jnp.float32) for _ in range(NBUF)]
            + [pltpu.SemaphoreType.DMA for _ in range(2 * NBUF)]
        ),
        compiler_params=pltpu.CompilerParams(
            use_tc_tiling_on_sc=True, needs_layout_passes=False),
    )
    def lookup(idx_hbm, table_hbm, out_hbm, idx_v, *refs):
        wbuf = refs[:NBUF]
        pbuf = refs[NBUF:2 * NBUF]
        wide = refs[2 * NBUF:3 * NBUF]
        comp = refs[3 * NBUF:4 * NBUF]
        gsem = refs[4 * NBUF:5 * NBUF]
        ssem = refs[5 * NBUF:]
        wid = lax.axis_index("s") * NUM_CORES + lax.axis_index("c")
        base = wid * b_per_w

        pltpu.sync_copy(idx_hbm.at[pl.ds(base, b_per_w)], idx_v)

        def prep_idx(i, b):
            @pl.loop(0, chunk // L)
            def _g(g):
                v = idx_v[pl.ds(i * chunk + g * L, L)]
                wbuf[b][pl.ds(g * L, L)] = lax.shift_right_logical(v, 1)
                pbuf[b][pl.ds(g * L, L)] = (v & 1) * DSZ

        def start_gather(b):
            pltpu.async_copy(table_hbm.at[wbuf[b]], wide[b], gsem[b])

        def wait_gather(b):
            pltpu.make_async_copy(
                table_hbm.at[pl.ds(0, chunk)], wide[b], gsem[b]).wait()

        def compact(b):
            @pl.loop(0, chunk // L)
            def _g(g):
                r0 = g * L
                rv = lax.iota(jnp.int32, L) + jnp.full((L,), r0, jnp.int32)
                par = pbuf[b][pl.ds(r0, L)]
                for c in range(DSZ):
                    vals = plsc.load_gather(wide[b], [rv, par + c])
                    plsc.store_scatter(comp[b], [rv, jnp.full((L,), c, jnp.int32)], vals)

        def start_store(i, b):
            pltpu.async_copy(comp[b], out_hbm.at[pl.ds(base + i * chunk, chunk)],
                             ssem[b])

        def wait_store(b):
            pltpu.make_async_copy(
                comp[b], out_hbm.at[pl.ds(0, chunk)], ssem[b]).wait()

        for b in range(NBUF):
            prep_idx(b, b)
            start_gather(b)

        @pl.loop(0, n_outer - 1)
        def _round(j):
            i0 = j * NBUF
            for b in range(NBUF):
                wait_gather(b)
                compact(b)
                start_store(i0 + b, b)
            for b in range(NBUF):
                wait_store(b)
                prep_idx(i0 + NBUF + b, b)
                start_gather(b)

        i0 = (n_outer - 1) * NBUF
        for b in range(NBUF):
            wait_gather(b)
            compact(b)
            start_store(i0 + b, b)
        for b in range(NBUF):
            wait_store(b)

    return lookup


def kernel(x, table):
    bsz, hist = x.shape
    vsz, dsz = table.shape
    flat = x.reshape(bsz * hist)
    wide_table = table.reshape(vsz // 2, 2 * dsz)
    lookup = _make_lookup(bsz * hist, vsz // 2, 128)
    out = lookup(flat, wide_table)
    return out.reshape(bsz, hist, dsz)


# speedup vs baseline: 1.8966x; 1.8966x over previous
import functools

import jax
import jax.numpy as jnp
from jax import lax
from jax.experimental import pallas as pl
from jax.experimental.pallas import tpu as pltpu
from jax.experimental.pallas import tpu_sc as plsc

NUM_CORES = 2
NUM_SUBCORES = 16
NUM_WORKERS = NUM_CORES * NUM_SUBCORES
NBUF = 2
L = 16
DSZ = 64


@functools.lru_cache(maxsize=None)
def _make_lookup(batch: int, vhalf: int, chunk: int):
    assert batch % NUM_WORKERS == 0
    b_per_w = batch // NUM_WORKERS
    assert b_per_w % (chunk * NBUF) == 0
    n_outer = b_per_w // (chunk * NBUF)

    mesh = plsc.VectorSubcoreMesh(core_axis_name="c", subcore_axis_name="s")

    @functools.partial(
        pl.kernel,
        mesh=mesh,
        out_type=jax.ShapeDtypeStruct((batch, DSZ), jnp.float32),
        scratch_types=(
            [pltpu.VMEM((chunk,), jnp.int32) for _ in range(NBUF)]
            + [pltpu.VMEM((chunk,), jnp.int32) for _ in range(NBUF)]
            + [pltpu.VMEM((chunk,), jnp.int32) for _ in range(NBUF)]
            + [pltpu.VMEM((chunk, 2 * DSZ), jnp.float32) for _ in range(NBUF)]
            + [pltpu.VMEM((chunk, DSZ), jnp.float32) for _ in range(NBUF)]
            + [pltpu.SemaphoreType.DMA for _ in range(2 * NBUF)]
        ),
        compiler_params=pltpu.CompilerParams(
            use_tc_tiling_on_sc=True, needs_layout_passes=False),
    )
    def lookup(idx_hbm, table_hbm, out_hbm, *refs):
        ibuf = refs[:NBUF]
        wbuf = refs[NBUF:2 * NBUF]
        sbuf = refs[2 * NBUF:3 * NBUF]
        wide = refs[3 * NBUF:4 * NBUF]
        comp = refs[4 * NBUF:5 * NBUF]
        gsem = refs[5 * NBUF:6 * NBUF]
        ssem = refs[6 * NBUF:]
        wid = lax.axis_index("s") * NUM_CORES + lax.axis_index("c")
        base = wid * b_per_w

        def prep_idx(i, b):
            pltpu.sync_copy(idx_hbm.at[pl.ds(base + i * chunk, chunk)], ibuf[b])

            @pl.loop(0, chunk // L)
            def _g(g):
                v = ibuf[b][pl.ds(g * L, L)]
                wbuf[b][pl.ds(g * L, L)] = lax.shift_right_logical(v, 1)
                sbuf[b][pl.ds(g * L, L)] = v & 1

        def start_gather(b):
            pltpu.async_copy(table_hbm.at[wbuf[b]], wide[b], gsem[b])

        def wait_gather(b):
            pltpu.make_async_copy(
                table_hbm.at[pl.ds(0, chunk)], wide[b], gsem[b]).wait()

        def compact(b):
            @pl.loop(0, chunk)
            def _row(r):
                rsplat = jnp.full((L,), 0, jnp.int32) + r
                parv = plsc.load_gather(sbuf[b], [rsplat])
                m = parv > 0
                for g in range(DSZ // L):
                    lo = wide[b][r, pl.ds(g * L, L)]
                    hi = wide[b][r, pl.ds(DSZ + g * L, L)]
                    comp[b][r, pl.ds(g * L, L)] = jnp.where(m, hi, lo)

        def start_store(i, b):
            pltpu.async_copy(comp[b], out_hbm.at[pl.ds(base + i * chunk, chunk)],
                             ssem[b])

        def wait_store(b):
            pltpu.make_async_copy(
                comp[b], out_hbm.at[pl.ds(0, chunk)], ssem[b]).wait()

        for b in range(NBUF):
            prep_idx(b, b)
            start_gather(b)

        @pl.loop(0, n_outer - 1)
        def _round(j):
            i0 = j * NBUF
            for b in range(NBUF):
                wait_gather(b)
                compact(b)
                start_store(i0 + b, b)
            for b in range(NBUF):
                wait_store(b)
                prep_idx(i0 + NBUF + b, b)
                start_gather(b)

        i0 = (n_outer - 1) * NBUF
        for b in range(NBUF):
            wait_gather(b)
            compact(b)
            start_store(i0 + b, b)
        for b in range(NBUF):
            wait_store(b)

    return lookup


def kernel(x, table):
    bsz, hist = x.shape
    vsz, dsz = table.shape
    flat = x.reshape(bsz * hist)
    wide_table = table.reshape(vsz // 2, 2 * dsz)
    lookup = _make_lookup(bsz * hist, vsz // 2, 160)
    out = lookup(flat, wide_table)
    return out.reshape(bsz, hist, dsz)


# speedup vs baseline: 1.9318x; 1.0185x over previous
import functools

import jax
import jax.numpy as jnp
from jax import lax
from jax.experimental import pallas as pl
from jax.experimental.pallas import tpu as pltpu
from jax.experimental.pallas import tpu_sc as plsc

NUM_CORES = 2
NUM_SUBCORES = 16
NUM_WORKERS = NUM_CORES * NUM_SUBCORES
NBUF = 2
L = 16
DSZ = 64


@functools.lru_cache(maxsize=None)
def _make_lookup(batch: int, vhalf: int, chunk: int):
    assert batch % NUM_WORKERS == 0
    b_per_w = batch // NUM_WORKERS
    assert b_per_w % (chunk * NBUF) == 0
    n_outer = b_per_w // (chunk * NBUF)

    mesh = plsc.VectorSubcoreMesh(core_axis_name="c", subcore_axis_name="s")

    @functools.partial(
        pl.kernel,
        mesh=mesh,
        out_type=jax.ShapeDtypeStruct((batch, DSZ), jnp.float32),
        scratch_types=(
            [pltpu.VMEM((chunk,), jnp.int32) for _ in range(NBUF)]
            + [pltpu.VMEM((chunk,), jnp.int32) for _ in range(NBUF)]
            + [pltpu.VMEM((chunk,), jnp.int32) for _ in range(NBUF)]
            + [pltpu.VMEM((chunk, 2 * DSZ), jnp.float32) for _ in range(NBUF)]
            + [pltpu.VMEM((chunk, DSZ), jnp.float32) for _ in range(NBUF)]
            + [pltpu.SemaphoreType.DMA for _ in range(2 * NBUF)]
        ),
        compiler_params=pltpu.CompilerParams(
            use_tc_tiling_on_sc=True, needs_layout_passes=False),
    )
    def lookup(idx_hbm, table_hbm, out_hbm, *refs):
        ibuf = refs[:NBUF]
        wbuf = refs[NBUF:2 * NBUF]
        sbuf = refs[2 * NBUF:3 * NBUF]
        wide = refs[3 * NBUF:4 * NBUF]
        comp = refs[4 * NBUF:5 * NBUF]
        gsem = refs[5 * NBUF:6 * NBUF]
        ssem = refs[6 * NBUF:]
        wid = lax.axis_index("s") * NUM_CORES + lax.axis_index("c")
        base = wid * b_per_w

        def prep_idx(i, b):
            pltpu.sync_copy(idx_hbm.at[pl.ds(base + i * chunk, chunk)], ibuf[b])

            @pl.loop(0, chunk // L)
            def _g(g):
                v = ibuf[b][pl.ds(g * L, L)]
                wbuf[b][pl.ds(g * L, L)] = lax.shift_right_logical(v, 1)
                sbuf[b][pl.ds(g * L, L)] = v & 1

        def start_gather(b):
            pltpu.async_copy(table_hbm.at[wbuf[b]], wide[b], gsem[b])

        def wait_gather(b):
            pltpu.make_async_copy(
                table_hbm.at[pl.ds(0, chunk)], wide[b], gsem[b]).wait()

        def compact(b):
            @pl.loop(0, chunk, unroll=8)
            def _row(r):
                rsplat = jnp.full((L,), 0, jnp.int32) + r
                parv = plsc.load_gather(sbuf[b], [rsplat])
                m = parv > 0
                for g in range(DSZ // L):
                    lo = wide[b][r, pl.ds(g * L, L)]
                    hi = wide[b][r, pl.ds(DSZ + g * L, L)]
                    comp[b][r, pl.ds(g * L, L)] = jnp.where(m, hi, lo)

        def start_store(i, b):
            pltpu.async_copy(comp[b], out_hbm.at[pl.ds(base + i * chunk, chunk)],
                             ssem[b])

        def wait_store(b):
            pltpu.make_async_copy(
                comp[b], out_hbm.at[pl.ds(0, chunk)], ssem[b]).wait()

        for b in range(NBUF):
            prep_idx(b, b)
            start_gather(b)

        @pl.loop(0, n_outer - 1)
        def _round(j):
            i0 = j * NBUF
            for b in range(NBUF):
                wait_gather(b)
                compact(b)
                start_store(i0 + b, b)
            for b in range(NBUF):
                wait_store(b)
                prep_idx(i0 + NBUF + b, b)
                start_gather(b)

        i0 = (n_outer - 1) * NBUF
        for b in range(NBUF):
            wait_gather(b)
            compact(b)
            start_store(i0 + b, b)
        for b in range(NBUF):
            wait_store(b)

    return lookup


def kernel(x, table):
    bsz, hist = x.shape
    vsz, dsz = table.shape
    flat = x.reshape(bsz * hist)
    wide_table = table.reshape(vsz // 2, 2 * dsz)
    lookup = _make_lookup(bsz * hist, vsz // 2, 160)
    out = lookup(flat, wide_table)
    return out.reshape(bsz, hist, dsz)
